# baseline (speedup 1.0000x reference)
import functools
import jax
import jax.numpy as jnp
from jax import lax
from jax.experimental import pallas as pl
from jax.experimental.pallas import tpu as pltpu
from jax.experimental.pallas import tpu_sc as plsc

N = 2048
G = 2
K = 64
NEG = -1e30

NC = 2
NS = 16
L = 16
NW = NC * NS
PTS = N // NW
EPW = N * K // NW


def _sc_mesh():
    return plsc.VectorSubcoreMesh(
        core_axis_name="c", subcore_axis_name="s", num_cores=NC,
        num_subcores=NS)


def _nbr_body(posx_h, posy_h, posz_h, batch_h, nbr1_h, nbr2_h, posx_v, posy_v,
              posz_v, batch_v, n1_v, n2_v):
    wid = lax.axis_index("s") * NC + lax.axis_index("c")
    base = wid * PTS
    pltpu.sync_copy(posx_h, posx_v)
    pltpu.sync_copy(posy_h, posy_v)
    pltpu.sync_copy(posz_h, posz_v)
    pltpu.sync_copy(batch_h, batch_v)
    lanes = lax.broadcasted_iota(jnp.int32, (L,), 0)
    zeros16 = jnp.zeros((L,), jnp.int32)

    def _cnt0(t, acc):
        b = batch_v[pl.ds(t * L, L)]
        inc = plsc.cumsum(jnp.where(b == 0, 1, 0))
        return acc + jnp.max(inc)
    s0 = lax.fori_loop(0, N // L, _cnt0, jnp.int32(0))

    def _point(p, carry):
        i = base + p
        i_v = jnp.full((L,), i, jnp.int32)
        px = plsc.load_gather(posx_v, [i_v])
        py = plsc.load_gather(posy_v, [i_v])
        pz = plsc.load_gather(posz_v, [i_v])
        bi = plsc.load_gather(batch_v, [i_v])
        bi_s = jnp.max(bi)
        gs = jnp.where(bi_s == 0, 0, s0)
        ge = jnp.where(bi_s == 0, s0, N)
        t_lo = gs // L
        t_hi = (ge + (L - 1)) // L
        pbase = p * K
        pb_v = jnp.full((L,), pbase, jnp.int32)

        def _chunk(t, cs):
            c1, c2 = cs
            j0 = t * L
            jv = lanes + j0
            xs = posx_v[pl.ds(j0, L)]
            ys = posy_v[pl.ds(j0, L)]
            zs = posz_v[pl.ds(j0, L)]
            bs = batch_v[pl.ds(j0, L)]
            dx = xs - px
            dy = ys - py
            dz = zs - pz
            d2 = dx * dx + dy * dy + dz * dz
            same = bs == bi
            v1 = (d2 <= 0.04) & same
            v2 = (d2 <= 0.16) & same
            packed = plsc.cumsum(jnp.where(v1, 1, 0)
                                 + jnp.where(v2, 65536, 0))
            inc1 = packed & 0xFFFF
            inc2 = lax.shift_right_logical(packed, 16)
            p1 = c1 + inc1 - 1
            p2 = c2 + inc2 - 1
            ok1 = v1 & (p1 < K)
            ok2 = v2 & (p2 < K)
            plsc.store_scatter(n1_v, [pb_v + p1], jv, mask=ok1)
            plsc.store_scatter(n2_v, [pb_v + p2], jv, mask=ok2)
            cmax = jnp.max(packed)
            c1 = c1 + (cmax & 0xFFFF)
            c2 = c2 + lax.shift_right_logical(cmax, 16)
            return c1, c2

        c1, c2 = lax.fori_loop(t_lo, t_hi, _chunk,
                               (jnp.int32(0), jnp.int32(0)))
        first1 = plsc.load_gather(n1_v, [pb_v])
        first2 = plsc.load_gather(n2_v, [pb_v])
        for q in range(K // L):
            sl = lanes + q * L
            idx = pb_v + sl
            cur1 = plsc.load_gather(n1_v, [idx])
            cur2 = plsc.load_gather(n2_v, [idx])
            plsc.store_scatter(n1_v, [idx], jnp.where(sl < c1, cur1, first1))
            plsc.store_scatter(n2_v, [idx], jnp.where(sl < c2, cur2, first2))
        return carry

    lax.fori_loop(0, PTS, _point, 0)
    pltpu.sync_copy(n1_v, nbr1_h.at[pl.ds(base * K, PTS * K)])
    pltpu.sync_copy(n2_v, nbr2_h.at[pl.ds(base * K, PTS * K)])


def _sc_neighbors(posx, posy, posz, batch):
    f = pl.kernel(
        _nbr_body,
        out_type=(jax.ShapeDtypeStruct((N * K,), jnp.int32),
                  jax.ShapeDtypeStruct((N * K,), jnp.int32)),
        mesh=_sc_mesh(),
        scratch_types=[
            pltpu.VMEM((N,), jnp.float32),
            pltpu.VMEM((N,), jnp.float32),
            pltpu.VMEM((N,), jnp.float32),
            pltpu.VMEM((N,), jnp.int32),
            pltpu.VMEM((PTS * K,), jnp.int32),
            pltpu.VMEM((PTS * K,), jnp.int32),
        ],
        compiler_params=pltpu.CompilerParams(needs_layout_passes=False),
    )
    return f(posx, posy, posz, batch)


def _gather_body(CH, D, table_h, idx_h, out_h, idx_v, rows0_v, rows1_v, sem0,
                 sem1):
    wid = lax.axis_index("s") * NC + lax.axis_index("c")
    base = wid * EPW
    pltpu.sync_copy(idx_h.at[pl.ds(base, EPW)], idx_v)
    nch = EPW // CH
    bufs = (rows0_v, rows1_v)
    sems = (sem0, sem1)

    def _start(c):
        return pltpu.async_copy(
            table_h.at[idx_v.at[pl.ds(c * CH, CH)]], bufs[c % 2], sems[c % 2])

    pending = _start(0)
    for c in range(nch):
        nxt = _start(c + 1) if c + 1 < nch else None
        pending.wait()
        pltpu.sync_copy(bufs[c % 2], out_h.at[pl.ds(base + c * CH, CH)])
        pending = nxt


def _sc_gather(table, idx, CH):
    D = table.shape[1]
    f = pl.kernel(
        functools.partial(_gather_body, CH, D),
        out_type=jax.ShapeDtypeStruct((N * K, D), jnp.float32),
        mesh=_sc_mesh(),
        scratch_types=[
            pltpu.VMEM((EPW,), jnp.int32),
            pltpu.VMEM((CH, D), jnp.float32),
            pltpu.VMEM((CH, D), jnp.float32),
            pltpu.SemaphoreType.DMA,
            pltpu.SemaphoreType.DMA,
        ],
        compiler_params=pltpu.CompilerParams(needs_layout_passes=False),
    )
    return f(table, idx)


def _prep_body(posp_ref, w11_ref, w2b_ref, w31b_ref, t1_ref, b2_ref, c3_ref):
    p = posp_ref[...]
    t1_ref[...] = jnp.dot(p, w11_ref[...], preferred_element_type=jnp.float32)
    b2_ref[...] = jnp.dot(p, w2b_ref[...], preferred_element_type=jnp.float32)
    c3_ref[...] = jnp.dot(p, w31b_ref[...], preferred_element_type=jnp.float32)


def _prep(posp, w11p, w2bp, w31bp):
    return pl.pallas_call(
        _prep_body,
        out_shape=(
            jax.ShapeDtypeStruct((N, 128), jnp.float32),
            jax.ShapeDtypeStruct((N, 128), jnp.float32),
            jax.ShapeDtypeStruct((N, 256), jnp.float32),
        ),
    )(posp, w11p, w2bp, w31bp)


def _sa_body(P, D, HO, WA, e_ref, tself_ref, b1_ref, w2_ref,
             b2_ref, w3_ref, b3_ref, wa_ref, addrows_ref, out_ref):
    e = e_ref[...]
    tself = tself_ref[...]
    trep = jnp.broadcast_to(tself[:, None, :], (P, K, D)).reshape(P * K, D)
    h = jax.nn.relu(e - trep + b1_ref[...])
    h = jax.nn.relu(jnp.dot(h, w2_ref[...], preferred_element_type=jnp.float32)
                    + b2_ref[...])
    h = jax.nn.relu(jnp.dot(h, w3_ref[...], preferred_element_type=jnp.float32)
                    + b3_ref[...])
    x = jnp.max(h.reshape(P, K, HO), axis=1)
    if WA:
        x = jnp.dot(x, wa_ref[...], preferred_element_type=jnp.float32) \
            + addrows_ref[...]
    out_ref[...] = x


def _sa_stage(e, tself, b1, w2, b2, w3, b3, wa, addrows, P):
    D = e.shape[1]
    HO = b3.shape[0]
    WA_FLAG = wa is not None
    OUT = wa.shape[1] if WA_FLAG else HO
    grid = (N // P,)
    if not WA_FLAG:
        wa = jnp.zeros((HO, HO), jnp.float32)
        addrows = jnp.zeros((N, HO), jnp.float32)
    body = functools.partial(_sa_body, P, D, HO, WA_FLAG)
    return pl.pallas_call(
        body,
        grid=grid,
        in_specs=[
            pl.BlockSpec((P * K, D), lambda i: (i, 0)),
            pl.BlockSpec((P, D), lambda i: (i, 0)),
            pl.BlockSpec(b1.shape, lambda i: (0,)),
            pl.BlockSpec(w2.shape, lambda i: (0, 0)),
            pl.BlockSpec(b2.shape, lambda i: (0,)),
            pl.BlockSpec(w3.shape, lambda i: (0, 0)),
            pl.BlockSpec(b3.shape, lambda i: (0,)),
            pl.BlockSpec(wa.shape, lambda i: (0, 0)),
            pl.BlockSpec((P, OUT), lambda i: (i, 0)),
        ],
        out_specs=pl.BlockSpec((P, OUT), lambda i: (i, 0)),
        out_shape=jax.ShapeDtypeStruct((N, OUT), jnp.float32),
    )(e, tself, b1, w2, b2, w3, b3, wa, addrows)


def _final_body(x2_ref, c3_ref, batch_ref, qpi_ref, w3a_ref, b31_ref, w32_ref,
                b32_ref, w33_ref, b33_ref, wp1_ref, bp1_ref, wp2_ref, bp2_ref,
                wp3_ref, bp3_ref, out_ref):
    x2 = x2_ref[...]
    h = jax.nn.relu(jnp.dot(x2, w3a_ref[...], preferred_element_type=jnp.float32)
                    + c3_ref[...] + b31_ref[...])
    h = jax.nn.relu(jnp.dot(h, w32_ref[...], preferred_element_type=jnp.float32)
                    + b32_ref[...])
    h = jax.nn.relu(jnp.dot(h, w33_ref[...], preferred_element_type=jnp.float32)
                    + b33_ref[...])
    bcol = batch_ref[...]
    s0 = jnp.max(jnp.where(bcol == 0, h, NEG), axis=0, keepdims=True)
    s1 = jnp.max(jnp.where(bcol == 1, h, NEG), axis=0, keepdims=True)
    scene = jnp.concatenate([s0, s1], axis=0)
    rio = jax.lax.broadcasted_iota(jnp.int32, (G, N), 1)
    q0 = qpi_ref[0]
    q1 = qpi_ref[1]
    qcol = jnp.concatenate(
        [jnp.full((1, 1), q0, jnp.int32), jnp.full((1, 1), q1, jnp.int32)], axis=0)
    oh = (rio == qcol).astype(jnp.float32)
    pf = jnp.dot(oh, x2, preferred_element_type=jnp.float32)
    ef = jnp.concatenate([pf, scene], axis=1)
    o = jax.nn.relu(jnp.dot(ef, wp1_ref[...], preferred_element_type=jnp.float32)
                    + bp1_ref[...])
    o = jax.nn.relu(jnp.dot(o, wp2_ref[...], preferred_element_type=jnp.float32)
                    + bp2_ref[...])
    o = jnp.dot(o, wp3_ref[...], preferred_element_type=jnp.float32) + bp3_ref[...]
    t = o[:, 0:3]
    r1 = o[:, 3:6]
    r2 = o[:, 6:9]
    r1 = r1 * jax.lax.rsqrt(jnp.sum(r1 * r1, axis=1, keepdims=True))
    r2 = r2 - jnp.sum(r1 * r2, axis=1, keepdims=True) * r1
    r2 = r2 * jax.lax.rsqrt(jnp.sum(r2 * r2, axis=1, keepdims=True))
    a1, a2, a3 = r1[:, 0:1], r1[:, 1:2], r1[:, 2:3]
    c1, c2, c3 = r2[:, 0:1], r2[:, 1:2], r2[:, 2:3]
    r3x = a2 * c3 - a3 * c2
    r3y = a3 * c1 - a1 * c3
    r3z = a1 * c2 - a2 * c1
    zero = jnp.zeros((G, 1), jnp.float32)
    one = jnp.ones((G, 1), jnp.float32)
    out_ref[...] = jnp.concatenate(
        [a1, c1, r3x, t[:, 0:1],
         a2, c2, r3y, t[:, 1:2],
         a3, c3, r3z, t[:, 2:3],
         zero, zero, zero, one], axis=1)


def _final(x2, c3, batch2d, qpi, w3a, b31, w32, b32, w33, b33, pred):
    wp1, bp1, wp2, bp2, wp3, bp3 = pred
    return pl.pallas_call(
        _final_body,
        in_specs=[pl.BlockSpec(memory_space=pltpu.VMEM)] * 3
        + [pl.BlockSpec(memory_space=pltpu.SMEM)]
        + [pl.BlockSpec(memory_space=pltpu.VMEM)] * 12,
        out_shape=jax.ShapeDtypeStruct((G, 16), jnp.float32),
    )(x2, c3, batch2d, qpi, w3a, b31, w32, b32, w33, b33,
      wp1, bp1, wp2, bp2, wp3, bp3)


def _neighbors_tmp(pos, batch):
    d2 = jnp.sum((pos[:, None, :] - pos[None, :, :]) ** 2, axis=-1)
    same = batch[:, None] == batch[None, :]
    col = jnp.arange(N, dtype=jnp.int32)
    out = []
    for r in (0.2, 0.4):
        valid = (d2 <= r * r) & same
        keyv = jnp.where(valid, col[None, :], col[None, :] + N)
        neg, _ = jax.lax.top_k(-keyv, K)
        nbr = -neg
        mask = nbr < N
        nbr = jnp.where(mask, nbr, nbr[:, 0:1])
        out.append(nbr)
    return out


def kernel(pos, batch_idx, query_point_idx, mlp1, mlp2, mlp3, pred):
    W11, b11, W12, b12, W13, b13 = mlp1
    W21, b21, W22, b22, W23, b23 = mlp2
    W31, b31, W32, b32, W33, b33 = mlp3

    posp = jnp.pad(pos, ((0, 0), (0, 5)))
    w11p = jnp.pad(W11, ((0, 5), (0, 64)))
    w2bp = jnp.pad(W21[128:], ((0, 5), (0, 0)))
    w31bp = jnp.pad(W31[256:], ((0, 5), (0, 0)))
    t1, b2mat, c3 = _prep(posp, w11p, w2bp, w31bp)

    nbr1f, nbr2f = _sc_neighbors(pos[:, 0], pos[:, 1], pos[:, 2], batch_idx)

    e1 = _sc_gather(t1, nbr1f, CH=256)
    b11p = jnp.pad(b11, (0, 64))
    w12p = jnp.pad(W12, ((0, 64), (0, 0)))
    a2 = _sa_stage(e1, t1, b11p, w12p, b12, W13, b13,
                   W21[:128], b2mat, P=128)

    e2 = _sc_gather(a2, nbr2f, CH=256)
    x2 = _sa_stage(e2, b2mat, b21, W22, b22, W23, b23, None, None, P=128)

    batch2d = batch_idx.reshape(N, 1)
    out = _final(x2, c3, batch2d, query_point_idx,
                 W31[:256], b31, W32, b32, W33, b33, pred)
    return out


# speedup vs baseline: 1.6838x; 1.6838x over previous
import functools
import jax
import jax.numpy as jnp
from jax import lax
from jax.experimental import pallas as pl
from jax.experimental.pallas import tpu as pltpu
from jax.experimental.pallas import tpu_sc as plsc

N = 2048
G = 2
K = 64
NEG = -1e30

NC = 2
NS = 16
L = 16
NW = NC * NS
PTS = N // NW
EPW = N * K // NW


def _sc_mesh():
    return plsc.VectorSubcoreMesh(
        core_axis_name="c", subcore_axis_name="s", num_cores=NC,
        num_subcores=NS)


def _nbr_body(posx_h, posy_h, posz_h, batch_h, nbr1_h, nbr2_h, posx_v, posy_v,
              posz_v, batch_v, n1_v, n2_v):
    wid = lax.axis_index("s") * NC + lax.axis_index("c")
    base = wid * PTS
    pltpu.sync_copy(posx_h, posx_v)
    pltpu.sync_copy(posy_h, posy_v)
    pltpu.sync_copy(posz_h, posz_v)
    pltpu.sync_copy(batch_h, batch_v)
    lanes = lax.broadcasted_iota(jnp.int32, (L,), 0)
    zeros16 = jnp.zeros((L,), jnp.int32)

    def _cnt0(t, acc):
        b = batch_v[pl.ds(t * L, L)]
        inc = plsc.cumsum(jnp.where(b == 0, 1, 0))
        return acc + jnp.max(inc)
    s0 = lax.fori_loop(0, N // L, _cnt0, jnp.int32(0))

    def _point(p, carry):
        i = base + p
        i_v = jnp.full((L,), i, jnp.int32)
        px = plsc.load_gather(posx_v, [i_v])
        py = plsc.load_gather(posy_v, [i_v])
        pz = plsc.load_gather(posz_v, [i_v])
        bi = plsc.load_gather(batch_v, [i_v])
        bi_s = jnp.max(bi)
        gs = jnp.where(bi_s == 0, 0, s0)
        ge = jnp.where(bi_s == 0, s0, N)
        t_lo = gs // L
        t_hi = (ge + (L - 1)) // L
        pbase = p * K
        pb_v = jnp.full((L,), pbase, jnp.int32)

        def _chunk(t, cs):
            c1, c2 = cs
            j0 = t * L
            jv = lanes + j0
            xs = posx_v[pl.ds(j0, L)]
            ys = posy_v[pl.ds(j0, L)]
            zs = posz_v[pl.ds(j0, L)]
            bs = batch_v[pl.ds(j0, L)]
            dx = xs - px
            dy = ys - py
            dz = zs - pz
            d2 = dx * dx + dy * dy + dz * dz
            same = bs == bi
            v1 = (d2 <= 0.04) & same
            v2 = (d2 <= 0.16) & same
            packed = plsc.cumsum(jnp.where(v1, 1, 0)
                                 + jnp.where(v2, 65536, 0))
            inc1 = packed & 0xFFFF
            inc2 = lax.shift_right_logical(packed, 16)
            p1 = c1 + inc1 - 1
            p2 = c2 + inc2 - 1
            ok1 = v1 & (p1 < K)
            ok2 = v2 & (p2 < K)
            plsc.store_scatter(n1_v, [pb_v + p1], jv, mask=ok1)
            plsc.store_scatter(n2_v, [pb_v + p2], jv, mask=ok2)
            cmax = jnp.max(packed)
            c1 = c1 + (cmax & 0xFFFF)
            c2 = c2 + lax.shift_right_logical(cmax, 16)
            return c1, c2

        c1, c2 = lax.fori_loop(t_lo, t_hi, _chunk,
                               (jnp.int32(0), jnp.int32(0)))
        first1 = plsc.load_gather(n1_v, [pb_v])
        first2 = plsc.load_gather(n2_v, [pb_v])
        for q in range(K // L):
            sl = lanes + q * L
            idx = pb_v + sl
            cur1 = plsc.load_gather(n1_v, [idx])
            cur2 = plsc.load_gather(n2_v, [idx])
            plsc.store_scatter(n1_v, [idx], jnp.where(sl < c1, cur1, first1))
            plsc.store_scatter(n2_v, [idx], jnp.where(sl < c2, cur2, first2))
        return carry

    lax.fori_loop(0, PTS, _point, 0)
    pltpu.sync_copy(n1_v, nbr1_h.at[pl.ds(base * K, PTS * K)])
    pltpu.sync_copy(n2_v, nbr2_h.at[pl.ds(base * K, PTS * K)])


def _sc_neighbors(posx, posy, posz, batch):
    f = pl.kernel(
        _nbr_body,
        out_type=(jax.ShapeDtypeStruct((N * K,), jnp.int32),
                  jax.ShapeDtypeStruct((N * K,), jnp.int32)),
        mesh=_sc_mesh(),
        scratch_types=[
            pltpu.VMEM((N,), jnp.float32),
            pltpu.VMEM((N,), jnp.float32),
            pltpu.VMEM((N,), jnp.float32),
            pltpu.VMEM((N,), jnp.int32),
            pltpu.VMEM((PTS * K,), jnp.int32),
            pltpu.VMEM((PTS * K,), jnp.int32),
        ],
        compiler_params=pltpu.CompilerParams(needs_layout_passes=False),
    )
    return f(posx, posy, posz, batch)


def _gather_body(CH, D, table_h, idx_h, out_h, idx_v, rows0_v, rows1_v,
                 tshared_v, sem0, sem1):
    wid = lax.axis_index("s") * NC + lax.axis_index("c")
    base = wid * EPW
    @pl.when(lax.axis_index("s") == 0)
    def _stage():
        pltpu.sync_copy(table_h, tshared_v)

    pltpu.sync_copy(idx_h.at[pl.ds(base, EPW)], idx_v)
    plsc.subcore_barrier()
    nch = EPW // CH
    bufs = (rows0_v, rows1_v)
    sems = (sem0, sem1)

    def _start(c):
        return pltpu.async_copy(
            tshared_v.at[idx_v.at[pl.ds(c * CH, CH)]], bufs[c % 2],
            sems[c % 2])

    pending = _start(0)
    for c in range(nch):
        nxt = _start(c + 1) if c + 1 < nch else None
        pending.wait()
        pltpu.sync_copy(bufs[c % 2], out_h.at[pl.ds(base + c * CH, CH)])
        pending = nxt


def _sc_gather(table, idx, CH):
    D = table.shape[1]
    f = pl.kernel(
        functools.partial(_gather_body, CH, D),
        out_type=jax.ShapeDtypeStruct((N * K, D), jnp.float32),
        mesh=_sc_mesh(),
        scratch_types=[
            pltpu.VMEM((EPW,), jnp.int32),
            pltpu.VMEM((CH, D), jnp.float32),
            pltpu.VMEM((CH, D), jnp.float32),
            pltpu.VMEM_SHARED((N, D), jnp.float32),
            pltpu.SemaphoreType.DMA,
            pltpu.SemaphoreType.DMA,
        ],
        compiler_params=pltpu.CompilerParams(needs_layout_passes=False),
    )
    return f(table, idx)


def _prep_body(posp_ref, w11_ref, w2b_ref, w31b_ref, t1_ref, b2_ref, c3_ref):
    p = posp_ref[...]
    t1_ref[...] = jnp.dot(p, w11_ref[...], preferred_element_type=jnp.float32)
    b2_ref[...] = jnp.dot(p, w2b_ref[...], preferred_element_type=jnp.float32)
    c3_ref[...] = jnp.dot(p, w31b_ref[...], preferred_element_type=jnp.float32)


def _prep(posp, w11p, w2bp, w31bp):
    return pl.pallas_call(
        _prep_body,
        out_shape=(
            jax.ShapeDtypeStruct((N, 128), jnp.float32),
            jax.ShapeDtypeStruct((N, 128), jnp.float32),
            jax.ShapeDtypeStruct((N, 256), jnp.float32),
        ),
    )(posp, w11p, w2bp, w31bp)


def _sa_body(P, D, HO, WA, e_ref, tself_ref, b1_ref, w2_ref,
             b2_ref, w3_ref, b3_ref, wa_ref, addrows_ref, out_ref):
    e = e_ref[...]
    tself = tself_ref[...]
    trep = jnp.broadcast_to(tself[:, None, :], (P, K, D)).reshape(P * K, D)
    h = jax.nn.relu(e - trep + b1_ref[...])
    h = jax.nn.relu(jnp.dot(h, w2_ref[...], preferred_element_type=jnp.float32)
                    + b2_ref[...])
    h = jax.nn.relu(jnp.dot(h, w3_ref[...], preferred_element_type=jnp.float32)
                    + b3_ref[...])
    x = jnp.max(h.reshape(P, K, HO), axis=1)
    if WA:
        x = jnp.dot(x, wa_ref[...], preferred_element_type=jnp.float32) \
            + addrows_ref[...]
    out_ref[...] = x


def _sa_stage(e, tself, b1, w2, b2, w3, b3, wa, addrows, P):
    D = e.shape[1]
    HO = b3.shape[0]
    WA_FLAG = wa is not None
    OUT = wa.shape[1] if WA_FLAG else HO
    grid = (N // P,)
    if not WA_FLAG:
        wa = jnp.zeros((HO, HO), jnp.float32)
        addrows = jnp.zeros((N, HO), jnp.float32)
    body = functools.partial(_sa_body, P, D, HO, WA_FLAG)
    return pl.pallas_call(
        body,
        grid=grid,
        in_specs=[
            pl.BlockSpec((P * K, D), lambda i: (i, 0)),
            pl.BlockSpec((P, D), lambda i: (i, 0)),
            pl.BlockSpec(b1.shape, lambda i: (0,)),
            pl.BlockSpec(w2.shape, lambda i: (0, 0)),
            pl.BlockSpec(b2.shape, lambda i: (0,)),
            pl.BlockSpec(w3.shape, lambda i: (0, 0)),
            pl.BlockSpec(b3.shape, lambda i: (0,)),
            pl.BlockSpec(wa.shape, lambda i: (0, 0)),
            pl.BlockSpec((P, OUT), lambda i: (i, 0)),
        ],
        out_specs=pl.BlockSpec((P, OUT), lambda i: (i, 0)),
        out_shape=jax.ShapeDtypeStruct((N, OUT), jnp.float32),
    )(e, tself, b1, w2, b2, w3, b3, wa, addrows)


def _final_body(x2_ref, c3_ref, batch_ref, qpi_ref, w3a_ref, b31_ref, w32_ref,
                b32_ref, w33_ref, b33_ref, wp1_ref, bp1_ref, wp2_ref, bp2_ref,
                wp3_ref, bp3_ref, out_ref):
    x2 = x2_ref[...]
    h = jax.nn.relu(jnp.dot(x2, w3a_ref[...], preferred_element_type=jnp.float32)
                    + c3_ref[...] + b31_ref[...])
    h = jax.nn.relu(jnp.dot(h, w32_ref[...], preferred_element_type=jnp.float32)
                    + b32_ref[...])
    h = jax.nn.relu(jnp.dot(h, w33_ref[...], preferred_element_type=jnp.float32)
                    + b33_ref[...])
    bcol = batch_ref[...]
    s0 = jnp.max(jnp.where(bcol == 0, h, NEG), axis=0, keepdims=True)
    s1 = jnp.max(jnp.where(bcol == 1, h, NEG), axis=0, keepdims=True)
    scene = jnp.concatenate([s0, s1], axis=0)
    rio = jax.lax.broadcasted_iota(jnp.int32, (G, N), 1)
    q0 = qpi_ref[0]
    q1 = qpi_ref[1]
    qcol = jnp.concatenate(
        [jnp.full((1, 1), q0, jnp.int32), jnp.full((1, 1), q1, jnp.int32)], axis=0)
    oh = (rio == qcol).astype(jnp.float32)
    pf = jnp.dot(oh, x2, preferred_element_type=jnp.float32)
    ef = jnp.concatenate([pf, scene], axis=1)
    o = jax.nn.relu(jnp.dot(ef, wp1_ref[...], preferred_element_type=jnp.float32)
                    + bp1_ref[...])
    o = jax.nn.relu(jnp.dot(o, wp2_ref[...], preferred_element_type=jnp.float32)
                    + bp2_ref[...])
    o = jnp.dot(o, wp3_ref[...], preferred_element_type=jnp.float32) + bp3_ref[...]
    t = o[:, 0:3]
    r1 = o[:, 3:6]
    r2 = o[:, 6:9]
    r1 = r1 * jax.lax.rsqrt(jnp.sum(r1 * r1, axis=1, keepdims=True))
    r2 = r2 - jnp.sum(r1 * r2, axis=1, keepdims=True) * r1
    r2 = r2 * jax.lax.rsqrt(jnp.sum(r2 * r2, axis=1, keepdims=True))
    a1, a2, a3 = r1[:, 0:1], r1[:, 1:2], r1[:, 2:3]
    c1, c2, c3 = r2[:, 0:1], r2[:, 1:2], r2[:, 2:3]
    r3x = a2 * c3 - a3 * c2
    r3y = a3 * c1 - a1 * c3
    r3z = a1 * c2 - a2 * c1
    zero = jnp.zeros((G, 1), jnp.float32)
    one = jnp.ones((G, 1), jnp.float32)
    out_ref[...] = jnp.concatenate(
        [a1, c1, r3x, t[:, 0:1],
         a2, c2, r3y, t[:, 1:2],
         a3, c3, r3z, t[:, 2:3],
         zero, zero, zero, one], axis=1)


def _final(x2, c3, batch2d, qpi, w3a, b31, w32, b32, w33, b33, pred):
    wp1, bp1, wp2, bp2, wp3, bp3 = pred
    return pl.pallas_call(
        _final_body,
        in_specs=[pl.BlockSpec(memory_space=pltpu.VMEM)] * 3
        + [pl.BlockSpec(memory_space=pltpu.SMEM)]
        + [pl.BlockSpec(memory_space=pltpu.VMEM)] * 12,
        out_shape=jax.ShapeDtypeStruct((G, 16), jnp.float32),
    )(x2, c3, batch2d, qpi, w3a, b31, w32, b32, w33, b33,
      wp1, bp1, wp2, bp2, wp3, bp3)


def _neighbors_tmp(pos, batch):
    d2 = jnp.sum((pos[:, None, :] - pos[None, :, :]) ** 2, axis=-1)
    same = batch[:, None] == batch[None, :]
    col = jnp.arange(N, dtype=jnp.int32)
    out = []
    for r in (0.2, 0.4):
        valid = (d2 <= r * r) & same
        keyv = jnp.where(valid, col[None, :], col[None, :] + N)
        neg, _ = jax.lax.top_k(-keyv, K)
        nbr = -neg
        mask = nbr < N
        nbr = jnp.where(mask, nbr, nbr[:, 0:1])
        out.append(nbr)
    return out


def kernel(pos, batch_idx, query_point_idx, mlp1, mlp2, mlp3, pred):
    W11, b11, W12, b12, W13, b13 = mlp1
    W21, b21, W22, b22, W23, b23 = mlp2
    W31, b31, W32, b32, W33, b33 = mlp3

    posp = jnp.pad(pos, ((0, 0), (0, 5)))
    w11p = jnp.pad(W11, ((0, 5), (0, 64)))
    w2bp = jnp.pad(W21[128:], ((0, 5), (0, 0)))
    w31bp = jnp.pad(W31[256:], ((0, 5), (0, 0)))
    t1, b2mat, c3 = _prep(posp, w11p, w2bp, w31bp)

    nbr1f, nbr2f = _sc_neighbors(pos[:, 0], pos[:, 1], pos[:, 2], batch_idx)

    e1 = _sc_gather(t1, nbr1f, CH=256)
    b11p = jnp.pad(b11, (0, 64))
    w12p = jnp.pad(W12, ((0, 64), (0, 0)))
    a2 = _sa_stage(e1, t1, b11p, w12p, b12, W13, b13,
                   W21[:128], b2mat, P=128)

    e2 = _sc_gather(a2, nbr2f, CH=256)
    x2 = _sa_stage(e2, b2mat, b21, W22, b22, W23, b23, None, None, P=128)

    batch2d = batch_idx.reshape(N, 1)
    out = _final(x2, c3, batch2d, query_point_idx,
                 W31[:256], b31, W32, b32, W33, b33, pred)
    return out
